# Initial kernel scaffold; baseline (speedup 1.0000x reference)
import jax
import jax.numpy as jnp
from jax.experimental import pallas as pl


def kernel(x, edge_index, W_msg, W_self, b):
    raise NotImplementedError("write your pallas kernel here")


# speedup vs baseline: 5.9660x; 5.9660x over previous
import functools

import jax
import jax.numpy as jnp
from jax import lax
from jax.experimental import pallas as pl
from jax.experimental.pallas import tpu as pltpu
from jax.experimental.pallas import tpu_sc as plsc

N = 10000
E = 320000
D = 128

NC = 2
NS = 16
NW = NC * NS
K = 128
CHUNKS = -(-E // (NW * K))
EPT = CHUNKS * K
E_PAD = EPT * NW
N_PAD = 10240
RPT = N_PAD // NS


def _sc_segment_sum(x, src_p, dst_p, zrow, zdeg):
  mesh = plsc.VectorSubcoreMesh(core_axis_name="c", subcore_axis_name="s")

  @functools.partial(
      pl.kernel,
      out_type=(
          jax.ShapeDtypeStruct((NC, N_PAD, D), jnp.float32),
          jax.ShapeDtypeStruct((NC, N_PAD), jnp.float32),
      ),
      mesh=mesh,
      scratch_types=[
          pltpu.VMEM((CHUNKS, K), jnp.int32),
          pltpu.VMEM((CHUNKS, K), jnp.int32),
          pltpu.VMEM((K, D), jnp.float32),
          pltpu.VMEM((K,), jnp.float32),
          pltpu.VMEM_SHARED((N_PAD, D), jnp.float32),
          pltpu.VMEM_SHARED((N_PAD,), jnp.float32),
          pltpu.SemaphoreType.DMA,
      ],
  )
  def k(x_hbm, src_hbm, dst_hbm, zrow_hbm, zdeg_hbm, agg_out, deg_out,
        srcv, dstv, rows, ones_v, agg_s, deg_s, sem):
    cid = lax.axis_index("c")
    sid = lax.axis_index("s")
    tid = cid * NS + sid

    pltpu.sync_copy(zrow_hbm, agg_s.at[pl.ds(sid * RPT, RPT)])
    pltpu.sync_copy(zdeg_hbm, deg_s.at[pl.ds(sid * RPT, RPT)])
    for i in range(K // 16):
      ones_v[pl.ds(i * 16, 16)] = jnp.ones((16,), jnp.float32)
    pltpu.sync_copy(src_hbm.at[tid], srcv)
    pltpu.sync_copy(dst_hbm.at[tid], dstv)
    plsc.subcore_barrier()

    def chunk(j, carry):
      pltpu.async_copy(x_hbm.at[srcv.at[j]], rows, sem).wait()
      pltpu.sync_copy(rows, agg_s.at[dstv.at[j]], add=True)
      pltpu.sync_copy(ones_v, deg_s.at[dstv.at[j]], add=True)
      return carry

    lax.fori_loop(0, CHUNKS, chunk, 0)
    plsc.subcore_barrier()

    sl = pl.ds(sid * RPT, RPT)
    pltpu.sync_copy(agg_s.at[sl], agg_out.at[cid].at[sl])
    pltpu.sync_copy(deg_s.at[sl], deg_out.at[cid].at[sl])

  return k(x, src_p, dst_p, zrow, zdeg)


def _tc_stats(deg_t):
  def k(deg_ref, out_ref):
    d = deg_ref[:, 0:1] + deg_ref[:, 1:2]
    row = lax.broadcasted_iota(jnp.int32, (N_PAD, 1), 0)
    ld = jnp.where(row < N, jnp.log(1.0 + d), 0.0)
    s1 = jnp.sum(ld)
    s2 = jnp.sum(ld * ld)
    mu = s1 / N
    var = jnp.maximum(s2 / N - mu * mu, 0.0)
    sd = jnp.sqrt(var)
    sd = jnp.where(sd > 0.0, sd, 1.0)
    col = lax.broadcasted_iota(jnp.int32, (1, 2), 1)
    out_ref[...] = jnp.where(col == 0, mu, sd)

  return pl.pallas_call(
      k, out_shape=jax.ShapeDtypeStruct((1, 2), jnp.float32))(deg_t)


BN = 400


def _tc_dense(agg, deg_t, musd, x, w_msg, w_self, b2):
  def k(agg_ref, deg_ref, musd_ref, x_ref, wm_ref, ws_ref, b_ref, out_ref):
    a = agg_ref[0] + agg_ref[1]
    d = deg_ref[:, 0:1] + deg_ref[:, 1:2]
    am = a / jnp.maximum(d, 1.0)
    h = (jnp.dot(am, wm_ref[...], preferred_element_type=jnp.float32)
         + jnp.dot(x_ref[...], ws_ref[...], preferred_element_type=jnp.float32)
         + b_ref[...])
    g = jnp.maximum(h, 0.0)
    mu = musd_ref[0, 0]
    sd = musd_ref[0, 1]
    nd = (jnp.log(1.0 + d) - mu) / sd
    out_ref[:, 0:1] = nd
    out_ref[:, 1:D + 1] = g

  grid = N // BN
  return pl.pallas_call(
      k,
      grid=(grid,),
      in_specs=[
          pl.BlockSpec((NC, BN, D), lambda i: (0, i, 0)),
          pl.BlockSpec((BN, 2), lambda i: (i, 0)),
          pl.BlockSpec((1, 2), lambda i: (0, 0), memory_space=pltpu.SMEM),
          pl.BlockSpec((BN, D), lambda i: (i, 0)),
          pl.BlockSpec((D, D), lambda i: (0, 0)),
          pl.BlockSpec((D, D), lambda i: (0, 0)),
          pl.BlockSpec((1, D), lambda i: (0, 0)),
      ],
      out_specs=pl.BlockSpec((BN, D + 1), lambda i: (i, 0)),
      out_shape=jax.ShapeDtypeStruct((N, D + 1), jnp.float32),
  )(agg, deg_t, musd, x, w_msg, w_self, b2)


def kernel(x, edge_index, W_msg, W_self, b):
  src = edge_index[0]
  dst = edge_index[1]
  pad = E_PAD - E
  src_p = jnp.concatenate(
      [src, jnp.zeros((pad,), jnp.int32)]).reshape(NW, CHUNKS, K)
  dst_p = jnp.concatenate(
      [dst, jnp.full((pad,), N, jnp.int32)]).reshape(NW, CHUNKS, K)
  zrow = jnp.zeros((RPT, D), jnp.float32)
  zdeg = jnp.zeros((RPT,), jnp.float32)

  agg, deg = _sc_segment_sum(x, src_p, dst_p, zrow, zdeg)
  deg_t = deg.T
  musd = _tc_stats(deg_t)
  return _tc_dense(agg, deg_t[:N], musd, x, W_msg, W_self, b.reshape(1, D))
